# baseline (speedup 1.0000x reference)
import functools

import jax
import jax.numpy as jnp
from jax import lax
from jax.experimental import pallas as pl
from jax.experimental.pallas import tpu as pltpu
from jax.experimental.pallas import tpu_sc as plsc

N_USERS = 25000
N_ITEMS = 25000
D = 64
NB_CLASSES = 10

N_HALF = 25000
ROWS_HALF = 25088
PAD_ROW = ROWS_HALF - 1
ROW_SHIFT = ROWS_HALF - N_HALF
N_PAD = 2 * ROWS_HALF
TILE_ROWS = ROWS_HALF // 16

CHUNK = 128
SUPER = 6
NBUF = 3
FIRE = NBUF - 1
NSUPER = 66
NPAIR = NSUPER // 2
EPT_CHUNKS = NSUPER * SUPER
E_PAD = 16 * EPT_CHUNKS * CHUNK

SEDGE = SUPER * CHUNK
CAPB = 6144
FLUSH = 3840
CAP_T = 57600
NTILE = 32
COMP_E = NTILE * CAP_T
COMP_ROWS = COMP_E // CHUNK

_mesh = plsc.VectorSubcoreMesh(core_axis_name="c", subcore_axis_name="s")


@functools.partial(
    pl.kernel,
    mesh=_mesh,
    out_type=[
        jax.ShapeDtypeStruct((COMP_E,), jnp.int32),
        jax.ShapeDtypeStruct((COMP_E,), jnp.int32),
        jax.ShapeDtypeStruct((COMP_E,), jnp.float32),
        jax.ShapeDtypeStruct((NTILE, 16), jnp.int32),
    ],
    scratch_types=[
        pltpu.VMEM((2, SUPER, CHUNK), jnp.int32),
        pltpu.VMEM((2, SUPER, CHUNK), jnp.int32),
        pltpu.VMEM((2, SUPER, CHUNK), jnp.float32),
        pltpu.VMEM((CAPB,), jnp.int32),
        pltpu.VMEM((CAPB,), jnp.int32),
        pltpu.VMEM((CAPB,), jnp.float32),
        pltpu.SemaphoreType.DMA,
    ],
    compiler_params=pltpu.CompilerParams(use_tc_tiling_on_sc=False, needs_layout_passes=False),
)
def _partition(src_hbm, dst_hbm, w_hbm, csrc_hbm, cidx_hbm, cw_hbm, cnt_hbm,
               meta_src, meta_dst, meta_w, bsrc, bidx, bw, sem_meta):
    c = lax.axis_index("c")
    s = lax.axis_index("s")
    lo = c * N_HALF
    tid = c * 16 + s
    base = tid * CAP_T
    cbase = s * EPT_CHUNKS

    def issue_meta(u, pb):
        rb = cbase + u * SUPER
        pltpu.async_copy(src_hbm.at[pl.ds(rb, SUPER)], meta_src.at[pb],
                         sem_meta)
        pltpu.async_copy(dst_hbm.at[pl.ds(rb, SUPER)], meta_dst.at[pb],
                         sem_meta)
        pltpu.async_copy(w_hbm.at[pl.ds(rb, SUPER)], meta_w.at[pb], sem_meta)

    def wait_meta(pb):
        pltpu.make_async_copy(src_hbm.at[pl.ds(0, SUPER)], meta_src.at[pb],
                              sem_meta).wait()
        pltpu.make_async_copy(dst_hbm.at[pl.ds(0, SUPER)], meta_dst.at[pb],
                              sem_meta).wait()
        pltpu.make_async_copy(w_hbm.at[pl.ds(0, SUPER)], meta_w.at[pb],
                              sem_meta).wait()

    def do_super(pb, pos, nf):
        for j in range(SUPER):
            def group_body(g, carry):
                p, = carry
                d16 = meta_dst[pb, j, pl.ds(g * 16, 16)]
                w16 = meta_w[pb, j, pl.ds(g * 16, 16)]
                s16 = meta_src[pb, j, pl.ds(g * 16, 16)]
                local = d16 - lo
                inr = (local >= 0) & (local < N_HALF)
                csum = plsc.cumsum(inr.astype(jnp.int32))
                dest = p + csum - 1
                plsc.store_scatter(bsrc, [dest], s16, mask=inr)
                plsc.store_scatter(bidx, [dest], local, mask=inr)
                plsc.store_scatter(bw, [dest], w16, mask=inr)
                return (p + jnp.max(csum),)

            (pos,) = lax.fori_loop(0, CHUNK // 16, group_body, (pos,))

            do_flush = pos >= FLUSH

            @pl.when(do_flush)
            def _():
                ob = base + nf * FLUSH
                pltpu.sync_copy(bsrc.at[pl.ds(0, FLUSH)],
                                csrc_hbm.at[pl.ds(ob, FLUSH)])
                pltpu.sync_copy(bidx.at[pl.ds(0, FLUSH)],
                                cidx_hbm.at[pl.ds(ob, FLUSH)])
                pltpu.sync_copy(bw.at[pl.ds(0, FLUSH)],
                                cw_hbm.at[pl.ds(ob, FLUSH)])
                for t in range(CHUNK // 16):
                    bsrc[pl.ds(t * 16, 16)] = bsrc[pl.ds(FLUSH + t * 16, 16)]
                    bidx[pl.ds(t * 16, 16)] = bidx[pl.ds(FLUSH + t * 16, 16)]
                    bw[pl.ds(t * 16, 16)] = bw[pl.ds(FLUSH + t * 16, 16)]

            pos = jnp.where(do_flush, pos - FLUSH, pos)
            nf = jnp.where(do_flush, nf + 1, nf)
        return pos, nf

    issue_meta(0, 0)
    wait_meta(0)
    issue_meta(1, 1)
    pos, nf = do_super(0, jnp.int32(0), jnp.int32(0))

    def pair_body(i, carry):
        pos, nf = carry
        wait_meta(1)
        issue_meta(2 * i + 2, 0)
        pos, nf = do_super(1, pos, nf)
        wait_meta(0)
        issue_meta(2 * i + 3, 1)
        pos, nf = do_super(0, pos, nf)
        return (pos, nf)

    pos, nf = lax.fori_loop(0, NPAIR - 1, pair_body, (pos, nf))
    wait_meta(1)
    pos, nf = do_super(1, pos, nf)

    zi = jnp.zeros((16,), jnp.int32)
    zf = jnp.zeros((16,), jnp.float32)
    pr = jnp.full((16,), PAD_ROW, jnp.int32)
    iota16 = lax.iota(jnp.int32, 16)
    for t in range(2 * SEDGE // 16):
        dst_i = pos + t * 16 + iota16
        plsc.store_scatter(bsrc, [dst_i], zi)
        plsc.store_scatter(bidx, [dst_i], pr)
        plsc.store_scatter(bw, [dst_i], zf)
    ob = base + nf * FLUSH
    pltpu.sync_copy(bsrc, csrc_hbm.at[pl.ds(ob, CAPB)])
    pltpu.sync_copy(bidx, cidx_hbm.at[pl.ds(ob, CAPB)])
    pltpu.sync_copy(bw, cw_hbm.at[pl.ds(ob, CAPB)])

    nsup = ((pos > 0).astype(jnp.int32)
            + (pos > SEDGE).astype(jnp.int32)
            + (pos > 2 * SEDGE).astype(jnp.int32)
            + (pos > 3 * SEDGE).astype(jnp.int32)
            + (pos > 4 * SEDGE).astype(jnp.int32)
            + nf * (FLUSH // SEDGE))
    nsup = nsup | 1
    bsrc[pl.ds(0, 16)] = jnp.full((16,), 1, jnp.int32) * nsup
    pltpu.sync_copy(bsrc.at[pl.ds(0, 16)], cnt_hbm.at[tid])


@functools.partial(
    pl.kernel,
    mesh=_mesh,
    out_type=jax.ShapeDtypeStruct((N_PAD, D), jnp.float32),
    scratch_types=[
        pltpu.VMEM((2, SUPER, CHUNK), jnp.int32),
        pltpu.VMEM((2, SUPER, CHUNK), jnp.int32),
        pltpu.VMEM((2, SUPER, CHUNK), jnp.float32),
        pltpu.VMEM((NBUF, CHUNK, D), jnp.float32),
        pltpu.VMEM((16,), jnp.int32),
        pltpu.VMEM_SHARED((ROWS_HALF, D), jnp.float32),
        pltpu.SemaphoreType.DMA,
        pltpu.SemaphoreType.DMA,
        pltpu.SemaphoreType.DMA,
    ],
    compiler_params=pltpu.CompilerParams(use_tc_tiling_on_sc=False, needs_layout_passes=False),
)
def _propagate(x_hbm, csrc_hbm, cidx_hbm, cw_hbm, cnt_hbm, out_hbm,
               meta_src, meta_idx, meta_w, rows_v, cntv,
               acc, sem_meta, sem_gather, sem_scatter):
    c = lax.axis_index("c")
    s = lax.axis_index("s")
    tid = c * 16 + s
    cbase = tid * (CAP_T // CHUNK)

    zvec = jnp.zeros((16,), jnp.float32)

    def zero_row(r, carry):
        for q in range(D // 16):
            rows_v[0, r, pl.ds(q * 16, 16)] = zvec
        return carry

    lax.fori_loop(0, CHUNK, zero_row, 0)
    rbase = s * TILE_ROWS
    for t in range(TILE_ROWS // CHUNK):
        pltpu.sync_copy(rows_v.at[0],
                        acc.at[pl.ds(rbase + t * CHUNK, CHUNK)])
    rem = TILE_ROWS - (TILE_ROWS // CHUNK) * CHUNK
    if rem:
        pltpu.sync_copy(
            rows_v.at[0, pl.ds(0, rem)],
            acc.at[pl.ds(rbase + (TILE_ROWS // CHUNK) * CHUNK, rem)])
    plsc.subcore_barrier()

    pltpu.sync_copy(cnt_hbm.at[tid], cntv)
    nsup = jnp.max(cntv[...])

    def issue_meta(u, pb):
        rb = cbase + u * SUPER
        pltpu.async_copy(csrc_hbm.at[pl.ds(rb, SUPER)], meta_src.at[pb],
                         sem_meta)
        pltpu.async_copy(cidx_hbm.at[pl.ds(rb, SUPER)], meta_idx.at[pb],
                         sem_meta)
        pltpu.async_copy(cw_hbm.at[pl.ds(rb, SUPER)], meta_w.at[pb],
                         sem_meta)

    def wait_meta(pb):
        pltpu.make_async_copy(csrc_hbm.at[pl.ds(0, SUPER)], meta_src.at[pb],
                              sem_meta).wait()
        pltpu.make_async_copy(cidx_hbm.at[pl.ds(0, SUPER)], meta_idx.at[pb],
                              sem_meta).wait()
        pltpu.make_async_copy(cw_hbm.at[pl.ds(0, SUPER)], meta_w.at[pb],
                              sem_meta).wait()

    def drain_scatter(b):
        pltpu.make_async_copy(rows_v.at[b], acc.at[meta_idx.at[0, 0]],
                              sem_scatter).wait()

    def fire_gather(pb, j):
        return pltpu.async_copy(x_hbm.at[meta_src.at[pb, j]],
                                rows_v.at[j % NBUF], sem_gather)

    def process_super(pb, first, refill):
        gathers = {}
        for j in range(FIRE):
            if not first:
                drain_scatter(j % NBUF)
            gathers[j] = fire_gather(pb, j)
        for j in range(SUPER):
            b = j % NBUF
            if j == 1 and refill is not None:
                nxt, pred = refill

                @pl.when(pred)
                def _():
                    issue_meta(nxt, 1 - pb)
            gathers[j].wait()

            def mul_body(g, carry):
                base = g * 16
                wm16 = meta_w[pb, j, pl.ds(base, 16)]
                for k in range(16):
                    r = base + k
                    ws = wm16[k]
                    vals = [rows_v[b, r, pl.ds(q * 16, 16)]
                            for q in range(D // 16)]
                    for q in range(D // 16):
                        rows_v[b, r, pl.ds(q * 16, 16)] = vals[q] * ws
                return carry

            lax.fori_loop(0, CHUNK // 16, mul_body, 0)
            pltpu.async_copy(rows_v.at[b], acc.at[meta_idx.at[pb, j]],
                             sem_scatter, add=True)
            if j + FIRE < SUPER:
                if not (first and j == 0):
                    drain_scatter((j + FIRE) % NBUF)
                gathers[j + FIRE] = fire_gather(pb, j + FIRE)

    P = nsup >> 1
    issue_meta(0, 0)
    wait_meta(0)

    @pl.when(nsup > 1)
    def _():
        issue_meta(1, 1)
    process_super(0, True, None)

    def pair_body(i, carry):
        wait_meta(1)
        process_super(1, False, (2 * i + 2, i >= 0))
        wait_meta(0)
        process_super(0, False, (2 * i + 3, i < P - 1))
        return carry

    lax.fori_loop(0, P, pair_body, 0)
    for b in range(NBUF):
        drain_scatter(b)
    plsc.subcore_barrier()

    pltpu.sync_copy(acc.at[pl.ds(rbase, TILE_ROWS)],
                    out_hbm.at[pl.ds(c * ROWS_HALF + rbase, TILE_ROWS)])


def _head_body(x0, x1, x2, x3, wp, bp, fin, logit):
    f = (x0[...] + x1[...] + x2[...] + x3[...]) * 0.25
    fin[...] = f
    logit[...] = (
        jnp.dot(f, wp[...], preferred_element_type=jnp.float32) + bp[...])


_HEAD_BLOCK = 512


def _head(x0, x1, x2, x3, wp, bp):
    grid = (N_PAD // _HEAD_BLOCK,)
    xspec = pl.BlockSpec((_HEAD_BLOCK, D), lambda i: (i, 0))
    return pl.pallas_call(
        _head_body,
        grid=grid,
        in_specs=[xspec, xspec, xspec, xspec,
                  pl.BlockSpec((D, 128), lambda i: (0, 0)),
                  pl.BlockSpec((1, 128), lambda i: (0, 0))],
        out_specs=[pl.BlockSpec((_HEAD_BLOCK, D), lambda i: (i, 0)),
                   pl.BlockSpec((_HEAD_BLOCK, 128), lambda i: (i, 0))],
        out_shape=[jax.ShapeDtypeStruct((N_PAD, D), jnp.float32),
                   jax.ShapeDtypeStruct((N_PAD, 128), jnp.float32)],
    )(x0, x1, x2, x3, wp, bp)


def kernel(user_emb, item_emb, edge_weight, W, b, edge_index):
    src = edge_index[0].astype(jnp.int32)
    dst = edge_index[1].astype(jnp.int32)
    src_p = src + jnp.where(src >= N_HALF, ROW_SHIFT, 0)

    pad_e = E_PAD - src.shape[0]
    src_p = jnp.pad(src_p, (0, pad_e)).reshape(-1, CHUNK)
    dst_p = jnp.pad(dst, (0, pad_e),
                    constant_values=N_USERS + N_ITEMS).reshape(-1, CHUNK)
    w_p = jnp.pad(edge_weight, (0, pad_e)).reshape(-1, CHUNK)

    csrc, cidx, cw, counts = _partition(src_p, dst_p, w_p)
    csrc = csrc.reshape(COMP_ROWS, CHUNK)
    cidx = cidx.reshape(COMP_ROWS, CHUNK)
    cw = cw.reshape(COMP_ROWS, CHUNK)

    x0 = jnp.zeros((N_PAD, D), jnp.float32)
    x0 = x0.at[:N_USERS].set(user_emb)
    x0 = x0.at[ROWS_HALF:ROWS_HALF + N_ITEMS].set(item_emb)

    x1 = _propagate(x0, csrc, cidx, cw, counts)
    x2 = _propagate(x1, csrc, cidx, cw, counts)
    x3 = _propagate(x2, csrc, cidx, cw, counts)

    wp = jnp.zeros((D, 128), jnp.float32).at[:, :NB_CLASSES].set(W)
    bp = jnp.zeros((1, 128), jnp.float32).at[0, :NB_CLASSES].set(b)
    fin, logit = _head(x0, x1, x2, x3, wp, bp)

    e_su = fin[:N_USERS]
    e_si = fin[ROWS_HALF:ROWS_HALF + N_ITEMS]
    su = logit[:N_USERS, :NB_CLASSES]
    si = logit[ROWS_HALF:ROWS_HALF + N_ITEMS, :NB_CLASSES]
    return (e_su, e_si, su, si)


# speedup vs baseline: 1.0244x; 1.0244x over previous
import functools

import jax
import jax.numpy as jnp
from jax import lax
from jax.experimental import pallas as pl
from jax.experimental.pallas import tpu as pltpu
from jax.experimental.pallas import tpu_sc as plsc

N_USERS = 25000
N_ITEMS = 25000
D = 64
NB_CLASSES = 10

N_HALF = 25000
ROWS_HALF = 25088
PAD_ROW = ROWS_HALF - 1
ROW_SHIFT = ROWS_HALF - N_HALF
N_PAD = 2 * ROWS_HALF
TILE_ROWS = ROWS_HALF // 16

CHUNK = 128
SUPER = 6
NBUF = 3
FIRE = NBUF - 1
NSUPER = 66
NPAIR = NSUPER // 2
EPT_CHUNKS = NSUPER * SUPER
E_PAD = 16 * EPT_CHUNKS * CHUNK

SEDGE = SUPER * CHUNK
CAPB = 6144
FLUSH = 3840
CAP_T = 57600
NTILE = 32
COMP_E = NTILE * CAP_T
COMP_ROWS = COMP_E // CHUNK

_mesh = plsc.VectorSubcoreMesh(core_axis_name="c", subcore_axis_name="s")


@functools.partial(
    pl.kernel,
    mesh=_mesh,
    out_type=[
        jax.ShapeDtypeStruct((COMP_E,), jnp.int32),
        jax.ShapeDtypeStruct((COMP_E,), jnp.int32),
        jax.ShapeDtypeStruct((COMP_E,), jnp.float32),
        jax.ShapeDtypeStruct((NTILE, 16), jnp.int32),
    ],
    scratch_types=[
        pltpu.VMEM((2, SUPER, CHUNK), jnp.int32),
        pltpu.VMEM((2, SUPER, CHUNK), jnp.int32),
        pltpu.VMEM((2, SUPER, CHUNK), jnp.float32),
        pltpu.VMEM((CAPB,), jnp.int32),
        pltpu.VMEM((CAPB,), jnp.int32),
        pltpu.VMEM((CAPB,), jnp.float32),
        pltpu.SemaphoreType.DMA,
    ],
    compiler_params=pltpu.CompilerParams(use_tc_tiling_on_sc=False, needs_layout_passes=False),
)
def _partition(src_hbm, dst_hbm, w_hbm, csrc_hbm, cidx_hbm, cw_hbm, cnt_hbm,
               meta_src, meta_dst, meta_w, bsrc, bidx, bw, sem_meta):
    c = lax.axis_index("c")
    s = lax.axis_index("s")
    lo = c * N_HALF
    tid = c * 16 + s
    base = tid * CAP_T
    cbase = s * EPT_CHUNKS

    def issue_meta(u, pb):
        rb = cbase + u * SUPER
        pltpu.async_copy(src_hbm.at[pl.ds(rb, SUPER)], meta_src.at[pb],
                         sem_meta)
        pltpu.async_copy(dst_hbm.at[pl.ds(rb, SUPER)], meta_dst.at[pb],
                         sem_meta)
        pltpu.async_copy(w_hbm.at[pl.ds(rb, SUPER)], meta_w.at[pb], sem_meta)

    def wait_meta(pb):
        pltpu.make_async_copy(src_hbm.at[pl.ds(0, SUPER)], meta_src.at[pb],
                              sem_meta).wait()
        pltpu.make_async_copy(dst_hbm.at[pl.ds(0, SUPER)], meta_dst.at[pb],
                              sem_meta).wait()
        pltpu.make_async_copy(w_hbm.at[pl.ds(0, SUPER)], meta_w.at[pb],
                              sem_meta).wait()

    def do_super(pb, pos, nf):
        for j in range(SUPER):
            def group_body(g, carry):
                p, = carry
                d16 = meta_dst[pb, j, pl.ds(g * 16, 16)]
                w16 = meta_w[pb, j, pl.ds(g * 16, 16)]
                s16 = meta_src[pb, j, pl.ds(g * 16, 16)]
                local = d16 - lo
                inr = (local >= 0) & (local < N_HALF)
                csum = plsc.cumsum(inr.astype(jnp.int32))
                dest = p + csum - 1
                plsc.store_scatter(bsrc, [dest], s16, mask=inr)
                plsc.store_scatter(bidx, [dest], local, mask=inr)
                plsc.store_scatter(bw, [dest], w16, mask=inr)
                return (p + jnp.max(csum),)

            (pos,) = lax.fori_loop(0, CHUNK // 16, group_body, (pos,))

            do_flush = pos >= FLUSH

            @pl.when(do_flush)
            def _():
                ob = base + nf * FLUSH
                pltpu.sync_copy(bsrc.at[pl.ds(0, FLUSH)],
                                csrc_hbm.at[pl.ds(ob, FLUSH)])
                pltpu.sync_copy(bidx.at[pl.ds(0, FLUSH)],
                                cidx_hbm.at[pl.ds(ob, FLUSH)])
                pltpu.sync_copy(bw.at[pl.ds(0, FLUSH)],
                                cw_hbm.at[pl.ds(ob, FLUSH)])
                for t in range(CHUNK // 16):
                    bsrc[pl.ds(t * 16, 16)] = bsrc[pl.ds(FLUSH + t * 16, 16)]
                    bidx[pl.ds(t * 16, 16)] = bidx[pl.ds(FLUSH + t * 16, 16)]
                    bw[pl.ds(t * 16, 16)] = bw[pl.ds(FLUSH + t * 16, 16)]

            pos = jnp.where(do_flush, pos - FLUSH, pos)
            nf = jnp.where(do_flush, nf + 1, nf)
        return pos, nf

    issue_meta(0, 0)
    wait_meta(0)
    issue_meta(1, 1)
    pos, nf = do_super(0, jnp.int32(0), jnp.int32(0))

    def pair_body(i, carry):
        pos, nf = carry
        wait_meta(1)
        issue_meta(2 * i + 2, 0)
        pos, nf = do_super(1, pos, nf)
        wait_meta(0)
        issue_meta(2 * i + 3, 1)
        pos, nf = do_super(0, pos, nf)
        return (pos, nf)

    pos, nf = lax.fori_loop(0, NPAIR - 1, pair_body, (pos, nf))
    wait_meta(1)
    pos, nf = do_super(1, pos, nf)

    zi = jnp.zeros((16,), jnp.int32)
    zf = jnp.zeros((16,), jnp.float32)
    pr = jnp.full((16,), PAD_ROW, jnp.int32)
    iota16 = lax.iota(jnp.int32, 16)
    for t in range(2 * SEDGE // 16):
        dst_i = pos + t * 16 + iota16
        plsc.store_scatter(bsrc, [dst_i], zi)
        plsc.store_scatter(bidx, [dst_i], pr)
        plsc.store_scatter(bw, [dst_i], zf)
    ob = base + nf * FLUSH
    pltpu.sync_copy(bsrc, csrc_hbm.at[pl.ds(ob, CAPB)])
    pltpu.sync_copy(bidx, cidx_hbm.at[pl.ds(ob, CAPB)])
    pltpu.sync_copy(bw, cw_hbm.at[pl.ds(ob, CAPB)])

    nsup = ((pos > 0).astype(jnp.int32)
            + (pos > SEDGE).astype(jnp.int32)
            + (pos > 2 * SEDGE).astype(jnp.int32)
            + (pos > 3 * SEDGE).astype(jnp.int32)
            + (pos > 4 * SEDGE).astype(jnp.int32)
            + nf * (FLUSH // SEDGE))
    nsup = nsup | 1
    bsrc[pl.ds(0, 16)] = jnp.full((16,), 1, jnp.int32) * nsup
    pltpu.sync_copy(bsrc.at[pl.ds(0, 16)], cnt_hbm.at[tid])


@functools.partial(
    pl.kernel,
    mesh=_mesh,
    out_type=jax.ShapeDtypeStruct((N_PAD, D), jnp.float32),
    scratch_types=[
        pltpu.VMEM((2, SUPER, CHUNK), jnp.int32),
        pltpu.VMEM((2, SUPER, CHUNK), jnp.int32),
        pltpu.VMEM((2, SUPER, CHUNK), jnp.float32),
        pltpu.VMEM((NBUF, CHUNK, D), jnp.float32),
        pltpu.VMEM((16,), jnp.int32),
        pltpu.VMEM_SHARED((ROWS_HALF, D), jnp.float32),
        pltpu.SemaphoreType.DMA,
        pltpu.SemaphoreType.DMA,
        pltpu.SemaphoreType.DMA,
    ],
    compiler_params=pltpu.CompilerParams(use_tc_tiling_on_sc=False, needs_layout_passes=False),
)
def _propagate(x_hbm, csrc_hbm, cidx_hbm, cw_hbm, cnt_hbm, out_hbm,
               meta_src, meta_idx, meta_w, rows_v, cntv,
               acc, sem_meta, sem_gather, sem_scatter):
    c = lax.axis_index("c")
    s = lax.axis_index("s")
    tid = c * 16 + s
    cbase = tid * (CAP_T // CHUNK)

    zvec = jnp.zeros((16,), jnp.float32)

    def zero_row(r, carry):
        for q in range(D // 16):
            rows_v[0, r, pl.ds(q * 16, 16)] = zvec
        return carry

    lax.fori_loop(0, CHUNK, zero_row, 0)
    rbase = s * TILE_ROWS
    for t in range(TILE_ROWS // CHUNK):
        pltpu.sync_copy(rows_v.at[0],
                        acc.at[pl.ds(rbase + t * CHUNK, CHUNK)])
    rem = TILE_ROWS - (TILE_ROWS // CHUNK) * CHUNK
    if rem:
        pltpu.sync_copy(
            rows_v.at[0, pl.ds(0, rem)],
            acc.at[pl.ds(rbase + (TILE_ROWS // CHUNK) * CHUNK, rem)])
    plsc.subcore_barrier()

    pltpu.sync_copy(cnt_hbm.at[tid], cntv)
    nsup = jnp.max(cntv[...])

    def issue_meta(u, pb):
        rb = cbase + u * SUPER
        pltpu.async_copy(csrc_hbm.at[pl.ds(rb, SUPER)], meta_src.at[pb],
                         sem_meta)
        pltpu.async_copy(cidx_hbm.at[pl.ds(rb, SUPER)], meta_idx.at[pb],
                         sem_meta)
        pltpu.async_copy(cw_hbm.at[pl.ds(rb, SUPER)], meta_w.at[pb],
                         sem_meta)

    def wait_meta(pb):
        pltpu.make_async_copy(csrc_hbm.at[pl.ds(0, SUPER)], meta_src.at[pb],
                              sem_meta).wait()
        pltpu.make_async_copy(cidx_hbm.at[pl.ds(0, SUPER)], meta_idx.at[pb],
                              sem_meta).wait()
        pltpu.make_async_copy(cw_hbm.at[pl.ds(0, SUPER)], meta_w.at[pb],
                              sem_meta).wait()

    def drain_scatter(b):
        pltpu.make_async_copy(rows_v.at[b], acc.at[meta_idx.at[0, 0]],
                              sem_scatter).wait()

    def fire_gather(pb, j):
        return pltpu.async_copy(x_hbm.at[meta_src.at[pb, j]],
                                rows_v.at[j % NBUF], sem_gather)

    def process_super(pb, first, refill):
        gathers = {}
        for j in range(FIRE):
            if not first:
                drain_scatter(j % NBUF)
            gathers[j] = fire_gather(pb, j)
        for j in range(SUPER):
            b = j % NBUF
            if j == 1 and refill is not None:
                nxt, pred = refill

                @pl.when(pred)
                def _():
                    issue_meta(nxt, 1 - pb)
            gathers[j].wait()

            def mul_body(g, carry):
                base = g * 16
                wm16 = meta_w[pb, j, pl.ds(base, 16)]
                for k4 in range(0, 16, 4):
                    rs = [base + k4 + e for e in range(4)]
                    wss = [wm16[k4 + e] for e in range(4)]
                    vals = [rows_v[b, rs[e], pl.ds(q * 16, 16)]
                            for e in range(4) for q in range(D // 16)]
                    for e in range(4):
                        for q in range(D // 16):
                            rows_v[b, rs[e], pl.ds(q * 16, 16)] = (
                                vals[e * (D // 16) + q] * wss[e])
                return carry

            lax.fori_loop(0, CHUNK // 16, mul_body, 0)
            pltpu.async_copy(rows_v.at[b], acc.at[meta_idx.at[pb, j]],
                             sem_scatter, add=True)
            if j + FIRE < SUPER:
                if not (first and j == 0):
                    drain_scatter((j + FIRE) % NBUF)
                gathers[j + FIRE] = fire_gather(pb, j + FIRE)

    P = nsup >> 1
    issue_meta(0, 0)
    wait_meta(0)

    @pl.when(nsup > 1)
    def _():
        issue_meta(1, 1)
    process_super(0, True, None)

    def pair_body(i, carry):
        wait_meta(1)
        process_super(1, False, (2 * i + 2, i >= 0))
        wait_meta(0)
        process_super(0, False, (2 * i + 3, i < P - 1))
        return carry

    lax.fori_loop(0, P, pair_body, 0)
    for b in range(NBUF):
        drain_scatter(b)
    plsc.subcore_barrier()

    pltpu.sync_copy(acc.at[pl.ds(rbase, TILE_ROWS)],
                    out_hbm.at[pl.ds(c * ROWS_HALF + rbase, TILE_ROWS)])


def _head_body(x0, x1, x2, x3, wp, bp, fin, logit):
    f = (x0[...] + x1[...] + x2[...] + x3[...]) * 0.25
    fin[...] = f
    logit[...] = (
        jnp.dot(f, wp[...], preferred_element_type=jnp.float32) + bp[...])


_HEAD_BLOCK = 512


def _head(x0, x1, x2, x3, wp, bp):
    grid = (N_PAD // _HEAD_BLOCK,)
    xspec = pl.BlockSpec((_HEAD_BLOCK, D), lambda i: (i, 0))
    return pl.pallas_call(
        _head_body,
        grid=grid,
        in_specs=[xspec, xspec, xspec, xspec,
                  pl.BlockSpec((D, 128), lambda i: (0, 0)),
                  pl.BlockSpec((1, 128), lambda i: (0, 0))],
        out_specs=[pl.BlockSpec((_HEAD_BLOCK, D), lambda i: (i, 0)),
                   pl.BlockSpec((_HEAD_BLOCK, 128), lambda i: (i, 0))],
        out_shape=[jax.ShapeDtypeStruct((N_PAD, D), jnp.float32),
                   jax.ShapeDtypeStruct((N_PAD, 128), jnp.float32)],
    )(x0, x1, x2, x3, wp, bp)


def kernel(user_emb, item_emb, edge_weight, W, b, edge_index):
    src = edge_index[0].astype(jnp.int32)
    dst = edge_index[1].astype(jnp.int32)
    src_p = src + jnp.where(src >= N_HALF, ROW_SHIFT, 0)

    pad_e = E_PAD - src.shape[0]
    src_p = jnp.pad(src_p, (0, pad_e)).reshape(-1, CHUNK)
    dst_p = jnp.pad(dst, (0, pad_e),
                    constant_values=N_USERS + N_ITEMS).reshape(-1, CHUNK)
    w_p = jnp.pad(edge_weight, (0, pad_e)).reshape(-1, CHUNK)

    csrc, cidx, cw, counts = _partition(src_p, dst_p, w_p)
    csrc = csrc.reshape(COMP_ROWS, CHUNK)
    cidx = cidx.reshape(COMP_ROWS, CHUNK)
    cw = cw.reshape(COMP_ROWS, CHUNK)

    x0 = jnp.zeros((N_PAD, D), jnp.float32)
    x0 = x0.at[:N_USERS].set(user_emb)
    x0 = x0.at[ROWS_HALF:ROWS_HALF + N_ITEMS].set(item_emb)

    x1 = _propagate(x0, csrc, cidx, cw, counts)
    x2 = _propagate(x1, csrc, cidx, cw, counts)
    x3 = _propagate(x2, csrc, cidx, cw, counts)

    wp = jnp.zeros((D, 128), jnp.float32).at[:, :NB_CLASSES].set(W)
    bp = jnp.zeros((1, 128), jnp.float32).at[0, :NB_CLASSES].set(b)
    fin, logit = _head(x0, x1, x2, x3, wp, bp)

    e_su = fin[:N_USERS]
    e_si = fin[ROWS_HALF:ROWS_HALF + N_ITEMS]
    su = logit[:N_USERS, :NB_CLASSES]
    si = logit[ROWS_HALF:ROWS_HALF + N_ITEMS, :NB_CLASSES]
    return (e_su, e_si, su, si)
